# Initial kernel scaffold; baseline (speedup 1.0000x reference)
import jax
import jax.numpy as jnp
from jax.experimental import pallas as pl


def kernel(logits, labels):
    raise NotImplementedError("write your pallas kernel here")


# speedup vs baseline: 8.9133x; 8.9133x over previous
import functools

import jax
import jax.numpy as jnp
from jax import lax
from jax.experimental import pallas as pl
from jax.experimental.pallas import tpu as pltpu
from jax.experimental.pallas import tpu_sc as plsc

THRESH = 0.7
N_MIN = 131072
IGNORE = 255

N, C, H, W = 8, 19, 512, 512
NPIX = N * H * W


_BH = 64


def _nll_pick_body(logits_ref, labels_ref, key_ref, nll_ref):
    lb = labels_ref[0]
    invalid = lb == IGNORE
    lb0 = jnp.where(invalid, 0, lb)

    x0 = logits_ref[0, 0]
    m = x0
    for c in range(1, C):
        m = jnp.maximum(m, logits_ref[0, c])

    s = jnp.zeros_like(m)
    xl = jnp.zeros_like(m)
    el = jnp.zeros_like(m)
    for c in range(C):
        xc = logits_ref[0, c]
        ec = jnp.exp(xc - m)
        s = s + ec
        sel = lb0 == c
        xl = xl + jnp.where(sel, xc, 0.0)
        el = el + jnp.where(sel, ec, 0.0)

    pick = el / s
    pick = jnp.where(invalid, 1.0, pick)
    nll = m + jnp.log(s) - xl
    key_ref[0] = lax.bitcast_convert_type(pick, jnp.int32)
    nll_ref[0] = nll


def _nll_pick(logits, labels):
    grid = (N, H // _BH)
    return pl.pallas_call(
        _nll_pick_body,
        grid=grid,
        in_specs=[
            pl.BlockSpec((1, C, _BH, W), lambda n, h: (n, 0, h, 0)),
            pl.BlockSpec((1, _BH, W), lambda n, h: (n, h, 0)),
        ],
        out_specs=[
            pl.BlockSpec((1, _BH, W), lambda n, h: (n, h, 0)),
            pl.BlockSpec((1, _BH, W), lambda n, h: (n, h, 0)),
        ],
        out_shape=[
            jax.ShapeDtypeStruct((N, H, W), jnp.int32),
            jax.ShapeDtypeStruct((N, H, W), jnp.float32),
        ],
    )(logits, labels)


_NW = 32
_PER_TILE = NPIX // _NW
_CHUNK = 4096
_NCHUNK = _PER_TILE // _CHUNK
_NVEC = _CHUNK // 16

def _wid():
    return lax.axis_index("s") * 2 + lax.axis_index("c")


def _lanes():
    return lax.iota(jnp.int32, 16)


def _zero_vmem(ref, n_words):
    z = jnp.zeros((16,), jnp.int32)

    def body(i, _):
        ref[pl.ds(i * 16, 16)] = z
        return 0

    lax.fori_loop(0, n_words // 16, body, 0)


def _hist_pass(keys_hbm, out_hbm, buf, hist, total, nbins, bin_fn, mask_fn):
    wid = _wid()
    base = wid * _PER_TILE
    lanes = _lanes()
    ones = jnp.full((16,), 1, jnp.int32)

    _zero_vmem(hist, nbins * 16)

    def chunk_body(ci, _):
        pltpu.sync_copy(keys_hbm.at[pl.ds(base + ci * _CHUNK, _CHUNK)], buf)

        def vec_body(i, _):
            key = buf[pl.ds(i * 16, 16)]
            idx = lanes * nbins + bin_fn(key)
            plsc.addupdate_scatter(hist, [idx], ones, mask=mask_fn(key))
            return 0

        lax.fori_loop(0, _NVEC, vec_body, 0)
        return 0

    lax.fori_loop(0, _NCHUNK, chunk_body, 0)

    def red_body(i, _):
        acc = hist[pl.ds(i * 16, 16)]
        for l in range(1, 16):
            acc = acc + hist[pl.ds(l * nbins + i * 16, 16)]
        total[pl.ds(i * 16, 16)] = acc
        return 0

    lax.fori_loop(0, nbins // 16, red_body, 0)
    pltpu.sync_copy(total, out_hbm.at[wid])


def _shr(key, amount):
    return lax.shift_right_logical(key, jnp.full((16,), amount, jnp.int32))


def _true_mask(key):
    return jnp.full((16,), True)


def _hist1_body(keys_hbm, out_hbm, buf, hist, total):
    _hist_pass(keys_hbm, out_hbm, buf, hist, total, 2048,
               lambda key: _shr(key, 21), _true_mask)


def _hist2_body(keys_hbm, sel_hbm, out_hbm, buf, hist, total, selbuf):
    pltpu.sync_copy(sel_hbm, selbuf)
    b1 = selbuf[0]

    def bin_fn(key):
        return jnp.bitwise_and(_shr(key, 10), jnp.full((16,), 0x7FF, jnp.int32))

    def mask_fn(key):
        return _shr(key, 21) == b1

    _hist_pass(keys_hbm, out_hbm, buf, hist, total, 2048, bin_fn, mask_fn)


def _hist3_body(keys_hbm, sel_hbm, out_hbm, buf, hist, total, selbuf):
    pltpu.sync_copy(sel_hbm, selbuf)
    p2 = selbuf[0]

    def bin_fn(key):
        return jnp.bitwise_and(key, jnp.full((16,), 0x3FF, jnp.int32))

    def mask_fn(key):
        return _shr(key, 10) == p2

    _hist_pass(keys_hbm, out_hbm, buf, hist, total, 1024, bin_fn, mask_fn)


def _merge_scan(h_hbm, rowbuf, acc, nbins, rank):
    _zero_vmem(acc, nbins)

    def row_body(r, _):
        pltpu.sync_copy(h_hbm.at[r], rowbuf)

        def add_body(i, _):
            acc[pl.ds(i * 16, 16)] = acc[pl.ds(i * 16, 16)] + rowbuf[pl.ds(i * 16, 16)]
            return 0

        lax.fori_loop(0, nbins // 16, add_body, 0)
        return 0

    lax.fori_loop(0, _NW, row_body, 0)

    def scan_body(i, carry):
        prefix, bcnt, cbef = carry
        v = acc[pl.ds(i * 16, 16)]
        cs = plsc.cumsum(v) + prefix
        le = cs <= rank
        bcnt = bcnt + jnp.sum(jnp.where(le, 1, 0))
        cbef = cbef + jnp.sum(jnp.where(le, v, 0))
        prefix = jnp.max(cs)
        return prefix, bcnt, cbef

    z = jnp.int32(0)
    _, b, cbefore = lax.fori_loop(0, nbins // 16, scan_body, (z, z, z))
    return b, rank - cbefore


def _splat_i32(x):
    return jnp.broadcast_to(x, (16,)).astype(jnp.int32)


def _merge1_body(h_hbm, sel_hbm, rowbuf, acc, selbuf):
    @pl.when(_wid() == 0)
    def _():
        b1, r1 = _merge_scan(h_hbm, rowbuf, acc, 2048, jnp.int32(N_MIN))
        selbuf[0] = _splat_i32(b1)
        selbuf[1] = _splat_i32(r1)
        pltpu.sync_copy(selbuf, sel_hbm)


def _merge2_body(h_hbm, sel1_hbm, sel_hbm, rowbuf, acc, selbuf, sel1buf):
    @pl.when(_wid() == 0)
    def _():
        pltpu.sync_copy(sel1_hbm, sel1buf)
        b1 = jnp.max(sel1buf[0])
        r1 = jnp.max(sel1buf[1])
        b2, r2 = _merge_scan(h_hbm, rowbuf, acc, 2048, r1)
        selbuf[0] = _splat_i32(b1 * 2048 + b2)
        selbuf[1] = _splat_i32(r2)
        pltpu.sync_copy(selbuf, sel_hbm)


def _merge3_body(h_hbm, sel2_hbm, kth_hbm, rowbuf, acc, kthbuf, sel2buf):
    @pl.when(_wid() == 0)
    def _():
        pltpu.sync_copy(sel2_hbm, sel2buf)
        p2 = jnp.max(sel2buf[0])
        r2 = jnp.max(sel2buf[1])
        b3, _r3 = _merge_scan(h_hbm, rowbuf, acc, 1024, r2)
        kthbuf[...] = _splat_i32(p2 * 1024 + b3)
        pltpu.sync_copy(kthbuf, kth_hbm)


@functools.cache
def _sc_kernels():
    mesh = plsc.VectorSubcoreMesh(core_axis_name="c", subcore_axis_name="s")
    cp = pltpu.CompilerParams(needs_layout_passes=False)
    i32, f32 = jnp.int32, jnp.float32
    hist1 = pl.kernel(
        _hist1_body, mesh=mesh, compiler_params=cp,
        out_type=jax.ShapeDtypeStruct((_NW, 2048), i32),
        scratch_types=[
            pltpu.VMEM((_CHUNK,), i32),
            pltpu.VMEM((2048 * 16,), i32),
            pltpu.VMEM((2048,), i32),
        ],
    )
    hist2 = pl.kernel(
        _hist2_body, mesh=mesh, compiler_params=cp,
        out_type=jax.ShapeDtypeStruct((_NW, 2048), i32),
        scratch_types=[
            pltpu.VMEM((_CHUNK,), i32),
            pltpu.VMEM((2048 * 16,), i32),
            pltpu.VMEM((2048,), i32),
            pltpu.VMEM((2, 16), i32),
        ],
    )
    hist3 = pl.kernel(
        _hist3_body, mesh=mesh, compiler_params=cp,
        out_type=jax.ShapeDtypeStruct((_NW, 1024), i32),
        scratch_types=[
            pltpu.VMEM((_CHUNK,), i32),
            pltpu.VMEM((1024 * 16,), i32),
            pltpu.VMEM((1024,), i32),
            pltpu.VMEM((2, 16), i32),
        ],
    )
    merge1 = pl.kernel(
        _merge1_body, mesh=mesh, compiler_params=cp,
        out_type=jax.ShapeDtypeStruct((2, 16), i32),
        scratch_types=[
            pltpu.VMEM((2048,), i32),
            pltpu.VMEM((2048,), i32),
            pltpu.VMEM((2, 16), i32),
        ],
    )
    merge2 = pl.kernel(
        _merge2_body, mesh=mesh, compiler_params=cp,
        out_type=jax.ShapeDtypeStruct((2, 16), i32),
        scratch_types=[
            pltpu.VMEM((2048,), i32),
            pltpu.VMEM((2048,), i32),
            pltpu.VMEM((2, 16), i32),
            pltpu.VMEM((2, 16), i32),
        ],
    )
    del f32
    merge3 = pl.kernel(
        _merge3_body, mesh=mesh, compiler_params=cp,
        out_type=jax.ShapeDtypeStruct((16,), i32),
        scratch_types=[
            pltpu.VMEM((1024,), i32),
            pltpu.VMEM((1024,), i32),
            pltpu.VMEM((16,), i32),
            pltpu.VMEM((2, 16), i32),
        ],
    )
    return hist1, merge1, hist2, merge2, hist3, merge3


_RBLK = 8
_RROWS = NPIX // 32768


_KEY_07 = 0x3F333333


def _loss_body(kth_ref, key_ref, nll_ref, out_ref, acc_ref):
    step = pl.program_id(0)

    @pl.when(step == 0)
    def _():
        acc_ref[0] = 0.0
        acc_ref[1] = 0.0

    thresh_key = jnp.maximum(kth_ref[0, 0], _KEY_07)
    k = key_ref[...]
    nl = nll_ref[...]
    keep = k <= thresh_key
    acc_ref[0] += jnp.sum(jnp.where(keep, nl, 0.0))
    acc_ref[1] += jnp.sum(keep.astype(jnp.float32))

    @pl.when(step == pl.num_programs(0) - 1)
    def _():
        out_ref[0, 0] = acc_ref[0] / jnp.maximum(acc_ref[1], 1.0)


def _masked_ce(kth, keys2d, nll2d):
    return pl.pallas_call(
        _loss_body,
        grid=(_RROWS // _RBLK,),
        in_specs=[
            pl.BlockSpec(memory_space=pltpu.SMEM),
            pl.BlockSpec((_RBLK, 32768), lambda i: (i, 0)),
            pl.BlockSpec((_RBLK, 32768), lambda i: (i, 0)),
        ],
        out_specs=pl.BlockSpec(memory_space=pltpu.SMEM),
        out_shape=jax.ShapeDtypeStruct((1, 1), jnp.float32),
        scratch_shapes=[pltpu.SMEM((2,), jnp.float32)],
    )(kth, keys2d, nll2d)


def kernel(logits, labels):
    hist1, merge1, hist2, merge2, hist3, merge3 = _sc_kernels()
    keys, nll = _nll_pick(logits, labels)
    keys_flat = keys.reshape(NPIX)
    h1 = hist1(keys_flat)
    sel1 = merge1(h1)
    h2 = hist2(keys_flat, sel1)
    sel2 = merge2(h2, sel1)
    h3 = hist3(keys_flat, sel2)
    kth = merge3(h3, sel2)
    loss = _masked_ce(kth[:1].reshape(1, 1),
                      keys.reshape(_RROWS, 32768),
                      nll.reshape(_RROWS, 32768))
    return loss.reshape(())


# speedup vs baseline: 14.8904x; 1.6706x over previous
import functools

import jax
import jax.numpy as jnp
from jax import lax
from jax.experimental import pallas as pl
from jax.experimental.pallas import tpu as pltpu
from jax.experimental.pallas import tpu_sc as plsc

THRESH = 0.7
N_MIN = 131072
IGNORE = 255

N, C, H, W = 8, 19, 512, 512
NPIX = N * H * W


_BH = 64


def _nll_pick_body(logits_ref, labels_ref, key_ref, nll_ref):
    lb = labels_ref[0]
    invalid = lb == IGNORE
    lb0 = jnp.where(invalid, 0, lb)

    x0 = logits_ref[0, 0]
    m = x0
    for c in range(1, C):
        m = jnp.maximum(m, logits_ref[0, c])

    s = jnp.zeros_like(m)
    xl = jnp.zeros_like(m)
    for c in range(C):
        xc = logits_ref[0, c]
        s = s + jnp.exp(xc - m)
        xl = xl + jnp.where(lb0 == c, xc, 0.0)

    pick = jnp.exp(xl - m) / s
    pick = jnp.where(invalid, 1.0, pick)
    nll = m + jnp.log(s) - xl
    key_ref[0] = lax.bitcast_convert_type(pick, jnp.int32)
    nll_ref[0] = nll


def _nll_pick(logits, labels):
    grid = (N, H // _BH)
    return pl.pallas_call(
        _nll_pick_body,
        grid=grid,
        in_specs=[
            pl.BlockSpec((1, C, _BH, W), lambda n, h: (n, 0, h, 0)),
            pl.BlockSpec((1, _BH, W), lambda n, h: (n, h, 0)),
        ],
        out_specs=[
            pl.BlockSpec((1, _BH, W), lambda n, h: (n, h, 0)),
            pl.BlockSpec((1, _BH, W), lambda n, h: (n, h, 0)),
        ],
        out_shape=[
            jax.ShapeDtypeStruct((N, H, W), jnp.int32),
            jax.ShapeDtypeStruct((N, H, W), jnp.float32),
        ],
    )(logits, labels)


_NW = 32
_ROWS = H // 4


def _wid():
    return lax.axis_index("s") * 2 + lax.axis_index("c")


def _lanes():
    return lax.iota(jnp.int32, 16)


def _zero_vmem(ref, n_words):
    z = jnp.zeros((16,), jnp.int32)

    def body(i, _):
        ref[pl.ds(i * 16, 16)] = z
        return 0

    lax.fori_loop(0, n_words // 16, body, 0, unroll=8)


def _hist_pass(keys_hbm, out_hbm, buf, hist, total, nbins, bin_fn, mask_fn):
    wid = _wid()
    img = wid >> 2
    quarter = wid & 3
    laneoff = _lanes() * nbins
    ones = jnp.full((16,), 1, jnp.int32)

    _zero_vmem(hist, nbins * 16)
    pltpu.sync_copy(keys_hbm.at[img, pl.ds(quarter * _ROWS, _ROWS), :], buf)

    def row_body(r, _):
        def vec_body(c, _):
            key = buf[r, pl.ds(c * 16, 16)]
            idx = laneoff + bin_fn(key)
            plsc.addupdate_scatter(hist, [idx], ones, mask=mask_fn(key))
            return 0

        lax.fori_loop(0, W // 16, vec_body, 0, unroll=8)
        return 0

    lax.fori_loop(0, _ROWS, row_body, 0)

    def red_body(i, _):
        acc = hist[pl.ds(i * 16, 16)]
        for l in range(1, 16):
            acc = acc + hist[pl.ds(l * nbins + i * 16, 16)]
        total[pl.ds(i * 16, 16)] = acc
        return 0

    lax.fori_loop(0, nbins // 16, red_body, 0)
    pltpu.sync_copy(total, out_hbm.at[wid])


def _shr(key, amount):
    return lax.shift_right_logical(key, jnp.full((16,), amount, jnp.int32))


def _true_mask(key):
    return jnp.full((16,), True)


def _hist1_body(keys_hbm, out_hbm, buf, hist, total):
    _hist_pass(keys_hbm, out_hbm, buf, hist, total, 2048,
               lambda key: _shr(key, 21), _true_mask)


def _hist2_body(keys_hbm, sel_hbm, out_hbm, buf, hist, total, selbuf):
    pltpu.sync_copy(sel_hbm, selbuf)
    b1 = selbuf[0]

    def bin_fn(key):
        return jnp.bitwise_and(_shr(key, 10), jnp.full((16,), 0x7FF, jnp.int32))

    def mask_fn(key):
        return _shr(key, 21) == b1

    _hist_pass(keys_hbm, out_hbm, buf, hist, total, 2048, bin_fn, mask_fn)


def _hist3_body(keys_hbm, sel_hbm, out_hbm, buf, hist, total, selbuf):
    pltpu.sync_copy(sel_hbm, selbuf)
    p2 = selbuf[0]

    def bin_fn(key):
        return jnp.bitwise_and(key, jnp.full((16,), 0x3FF, jnp.int32))

    def mask_fn(key):
        return _shr(key, 10) == p2

    _hist_pass(keys_hbm, out_hbm, buf, hist, total, 1024, bin_fn, mask_fn)


def _merge_scan(h_hbm, hbuf, nbins, rank):
    pltpu.sync_copy(h_hbm, hbuf)

    def scan_body(i, carry):
        prefix, bcnt, cbef = carry
        v = hbuf[0, pl.ds(i * 16, 16)]
        for r in range(1, _NW):
            v = v + hbuf[r, pl.ds(i * 16, 16)]
        cs = plsc.cumsum(v) + prefix
        le = cs <= rank
        bcnt = bcnt + jnp.sum(jnp.where(le, 1, 0))
        cbef = cbef + jnp.sum(jnp.where(le, v, 0))
        prefix = jnp.max(cs)
        return prefix, bcnt, cbef

    z = jnp.int32(0)
    _, b, cbefore = lax.fori_loop(0, nbins // 16, scan_body, (z, z, z))
    return b, rank - cbefore


def _splat_i32(x):
    return jnp.broadcast_to(x, (16,)).astype(jnp.int32)


def _merge1_body(h_hbm, sel_hbm, hbuf, selbuf):
    @pl.when(_wid() == 0)
    def _():
        b1, r1 = _merge_scan(h_hbm, hbuf, 2048, jnp.int32(N_MIN))
        selbuf[0] = _splat_i32(b1)
        selbuf[1] = _splat_i32(r1)
        pltpu.sync_copy(selbuf, sel_hbm)


def _merge2_body(h_hbm, sel1_hbm, sel_hbm, hbuf, selbuf, sel1buf):
    @pl.when(_wid() == 0)
    def _():
        pltpu.sync_copy(sel1_hbm, sel1buf)
        b1 = jnp.max(sel1buf[0])
        r1 = jnp.max(sel1buf[1])
        b2, r2 = _merge_scan(h_hbm, hbuf, 2048, r1)
        selbuf[0] = _splat_i32(b1 * 2048 + b2)
        selbuf[1] = _splat_i32(r2)
        pltpu.sync_copy(selbuf, sel_hbm)


def _merge3_body(h_hbm, sel2_hbm, kth_hbm, hbuf, kthbuf, sel2buf):
    @pl.when(_wid() == 0)
    def _():
        pltpu.sync_copy(sel2_hbm, sel2buf)
        p2 = jnp.max(sel2buf[0])
        r2 = jnp.max(sel2buf[1])
        b3, _r3 = _merge_scan(h_hbm, hbuf, 1024, r2)
        kthbuf[...] = _splat_i32(p2 * 1024 + b3)
        pltpu.sync_copy(kthbuf, kth_hbm)


@functools.cache
def _sc_kernels():
    mesh = plsc.VectorSubcoreMesh(core_axis_name="c", subcore_axis_name="s")
    cp = pltpu.CompilerParams(needs_layout_passes=False)
    i32 = jnp.int32
    hist1 = pl.kernel(
        _hist1_body, mesh=mesh, compiler_params=cp,
        out_type=jax.ShapeDtypeStruct((_NW, 2048), i32),
        scratch_types=[
            pltpu.VMEM((_ROWS, W), i32),
            pltpu.VMEM((2048 * 16,), i32),
            pltpu.VMEM((2048,), i32),
        ],
    )
    hist2 = pl.kernel(
        _hist2_body, mesh=mesh, compiler_params=cp,
        out_type=jax.ShapeDtypeStruct((_NW, 2048), i32),
        scratch_types=[
            pltpu.VMEM((_ROWS, W), i32),
            pltpu.VMEM((2048 * 16,), i32),
            pltpu.VMEM((2048,), i32),
            pltpu.VMEM((2, 16), i32),
        ],
    )
    hist3 = pl.kernel(
        _hist3_body, mesh=mesh, compiler_params=cp,
        out_type=jax.ShapeDtypeStruct((_NW, 1024), i32),
        scratch_types=[
            pltpu.VMEM((_ROWS, W), i32),
            pltpu.VMEM((1024 * 16,), i32),
            pltpu.VMEM((1024,), i32),
            pltpu.VMEM((2, 16), i32),
        ],
    )
    merge1 = pl.kernel(
        _merge1_body, mesh=mesh, compiler_params=cp,
        out_type=jax.ShapeDtypeStruct((2, 16), i32),
        scratch_types=[
            pltpu.VMEM((_NW, 2048), i32),
            pltpu.VMEM((2, 16), i32),
        ],
    )
    merge2 = pl.kernel(
        _merge2_body, mesh=mesh, compiler_params=cp,
        out_type=jax.ShapeDtypeStruct((2, 16), i32),
        scratch_types=[
            pltpu.VMEM((_NW, 2048), i32),
            pltpu.VMEM((2, 16), i32),
            pltpu.VMEM((2, 16), i32),
        ],
    )
    merge3 = pl.kernel(
        _merge3_body, mesh=mesh, compiler_params=cp,
        out_type=jax.ShapeDtypeStruct((16,), i32),
        scratch_types=[
            pltpu.VMEM((_NW, 1024), i32),
            pltpu.VMEM((16,), i32),
            pltpu.VMEM((2, 16), i32),
        ],
    )
    return hist1, merge1, hist2, merge2, hist3, merge3


_KEY_07 = 0x3F333333


def _loss_body(kth_ref, key_ref, nll_ref, out_ref, acc_ref):
    step = pl.program_id(0)

    @pl.when(step == 0)
    def _():
        acc_ref[0] = 0.0
        acc_ref[1] = 0.0

    thresh_key = jnp.maximum(kth_ref[0, 0], _KEY_07)
    k = key_ref[0]
    nl = nll_ref[0]
    keep = k <= thresh_key
    acc_ref[0] += jnp.sum(jnp.where(keep, nl, 0.0))
    acc_ref[1] += jnp.sum(keep.astype(jnp.float32))

    @pl.when(step == pl.num_programs(0) - 1)
    def _():
        out_ref[0, 0] = acc_ref[0] / jnp.maximum(acc_ref[1], 1.0)


def _masked_ce(kth, keys, nll):
    return pl.pallas_call(
        _loss_body,
        grid=(N,),
        in_specs=[
            pl.BlockSpec(memory_space=pltpu.SMEM),
            pl.BlockSpec((1, H, W), lambda i: (i, 0, 0)),
            pl.BlockSpec((1, H, W), lambda i: (i, 0, 0)),
        ],
        out_specs=pl.BlockSpec(memory_space=pltpu.SMEM),
        out_shape=jax.ShapeDtypeStruct((1, 1), jnp.float32),
        scratch_shapes=[pltpu.SMEM((2,), jnp.float32)],
    )(kth, keys, nll)


def kernel(logits, labels):
    hist1, merge1, hist2, merge2, hist3, merge3 = _sc_kernels()
    keys, nll = _nll_pick(logits, labels)
    h1 = hist1(keys)
    sel1 = merge1(h1)
    h2 = hist2(keys, sel1)
    sel2 = merge2(h2, sel1)
    h3 = hist3(keys, sel2)
    kth = merge3(h3, sel2)
    loss = _masked_ce(kth.reshape(1, 16), keys, nll)
    return loss.reshape(())
